# baseline (speedup 1.0000x reference)
import jax
import jax.numpy as jnp
from jax.experimental import pallas as pl
from jax.experimental.pallas import tpu as pltpu

_N = 1024
_D = 256
_T = 4096
_LAM = 0.25


def _dot(lhs, rhs):
    return jax.lax.dot_general(
        lhs, rhs, (((1,), (1,)), ((), ())),
        preferred_element_type=jnp.float32)


def _body(ts_ref, W_ref, h_ref, out_ref, tsn_s, W_s, h_s, nrm_s):
    W_s[...] = W_ref[...]
    h_s[...] = h_ref[...]
    ts = ts_ref[...]
    tsn_s[...] = ts / jnp.sqrt(jnp.sum(ts * ts, axis=1, keepdims=True))
    Wv = W_s[...]
    nrm_s[...] = jnp.sqrt(jnp.sum(Wv * Wv, axis=1)).reshape(8, 128)

    ridx = jax.lax.broadcasted_iota(jnp.int32, (8, 128), 0)
    cidx = jax.lax.broadcasted_iota(jnp.int32, (8, 128), 1)
    flat = ridx * 128 + cidx

    dot0 = _dot(tsn_s[pl.ds(0, 1), :], W_s[...]).reshape(8, 128)

    def step(t, carry):
        s, r_prev, c_prev, nn_prev, hp_prev, dot_stale = carry
        oh_prev = flat == r_prev
        nrm = jnp.where(oh_prev, nn_prev, nrm_s[...])
        nrm_s[...] = nrm
        h = jnp.where(oh_prev, hp_prev, h_s[...])
        h_s[...] = h
        dotv = jnp.where(oh_prev, c_prev, dot_stale)
        beta = dotv / nrm
        gain = jnp.exp(_LAM * (1.0 - 1024.0 * h / s))
        score = gain * beta
        n_star = jnp.argmax(score).astype(jnp.int32)
        onehot = flat == n_star
        bstar = jnp.sum(jnp.where(onehot, beta, 0.0))
        hstar = jnp.sum(jnp.where(onehot, h, 0.0))
        alpha = 0.01 / (1.0 + hstar / 20000.0)
        tsn = tsn_s[pl.ds(t, 1), :]
        tnext = jnp.minimum(t + 1, _T - 1)
        tsn_next = tsn_s[pl.ds(tnext, 1), :]
        dot_next = _dot(tsn_next, W_s[...]).reshape(8, 128)
        Ck = W_s[pl.ds(n_star, 1), :]
        new_row = Ck + alpha * bstar * (tsn - Ck)
        W_s[pl.ds(n_star, 1), :] = new_row
        corr = _dot(tsn_next, jnp.broadcast_to(new_row, (8, _D)))
        c_next = jnp.max(corr)
        nn = jnp.sqrt(jnp.sum(new_row * new_row))
        return (s + 1.0, n_star, c_next, nn, hstar + 1.0, dot_next)

    s0 = jnp.sum(h_ref[...])
    carry = (s0, jnp.int32(-1), 0.0, 1.0, 1.0, dot0)
    _, n_last, _, _, _, _ = jax.lax.fori_loop(0, _T, step, carry, unroll=2)
    out_ref[0, 0] = n_last


def kernel(all_ts, W, cumhisto):
    h2 = cumhisto.reshape(8, 128)
    out = pl.pallas_call(
        _body,
        out_shape=jax.ShapeDtypeStruct((1, 1), jnp.int32),
        in_specs=[
            pl.BlockSpec(memory_space=pltpu.VMEM),
            pl.BlockSpec(memory_space=pltpu.VMEM),
            pl.BlockSpec(memory_space=pltpu.VMEM),
        ],
        out_specs=pl.BlockSpec(memory_space=pltpu.SMEM),
        scratch_shapes=[
            pltpu.VMEM((_T, _D), jnp.float32),
            pltpu.VMEM((_N, _D), jnp.float32),
            pltpu.VMEM((8, 128), jnp.float32),
            pltpu.VMEM((8, 128), jnp.float32),
        ],
    )(all_ts, W, h2)
    return out[0, 0]


# speedup vs baseline: 1.0822x; 1.0822x over previous
import jax
import jax.numpy as jnp
from jax.experimental import pallas as pl
from jax.experimental.pallas import tpu as pltpu

_N = 1024
_D = 256
_T = 4096
_LAM = 0.25


def _dot(lhs, rhs):
    return jax.lax.dot_general(
        lhs, rhs, (((1,), (1,)), ((), ())),
        preferred_element_type=jnp.float32)


def _body(ts_ref, W_ref, h_ref, out_ref, tsn_s, W_s, h_s, nrm_s):
    W_s[...] = W_ref[...]
    h_s[...] = h_ref[...]
    ts = ts_ref[...]
    tsn_s[...] = ts / jnp.sqrt(jnp.sum(ts * ts, axis=1, keepdims=True))
    Wv = W_s[...]
    nrm_s[...] = jnp.sqrt(jnp.sum(Wv * Wv, axis=1)).reshape(8, 128)

    ridx = jax.lax.broadcasted_iota(jnp.int32, (8, 128), 0)
    cidx = jax.lax.broadcasted_iota(jnp.int32, (8, 128), 1)
    flat = ridx * 128 + cidx
    neg = jnp.float32(-jnp.inf)

    s0 = jnp.sum(h_ref[...])
    h0 = h_s[...]
    nrm0 = nrm_s[...]
    beta1 = _dot(tsn_s[pl.ds(0, 1), :], W_s[...]).reshape(8, 128) / nrm0
    gain1 = jnp.exp(_LAM * (1.0 - 1024.0 * h0 / s0))
    score1 = gain1 * beta1
    m1 = jnp.max(score1)
    i1 = jnp.argmax(score1).astype(jnp.int32)
    oh1 = flat == i1
    bstar1 = jnp.sum(jnp.where(oh1, beta1, 0.0))
    hstar1 = jnp.sum(jnp.where(oh1, h0, 0.0))
    Ck1 = W_s[pl.ds(i1, 1), :]
    dot1 = _dot(tsn_s[pl.ds(1, 1), :], W_s[...]).reshape(8, 128)
    corr0 = jnp.full((1, 8), neg, jnp.float32)

    def step(t, carry):
        (s, r_prev, nn_prev, hp_prev, gp, npr, corr,
         m1, i1, bstar1, hstar1, Ck1, dot_stale) = carry
        c = jnp.max(corr)
        beta_p = c / nn_prev
        score_p = gp * beta_p
        take_prev = (score_p > m1) | ((score_p == m1) & (r_prev < i1))
        n_star = jnp.where(take_prev, r_prev, i1)
        bstar = jnp.where(take_prev, beta_p, bstar1)
        hstar = jnp.where(take_prev, hp_prev, hstar1)
        alpha = 0.01 / (1.0 + hstar / 20000.0)
        tsn = tsn_s[pl.ds(t, 1), :]
        Ck = jnp.where(take_prev, npr, Ck1)
        new_row = Ck + alpha * bstar * (tsn - Ck)
        W_s[pl.ds(n_star, 1), :] = new_row
        t1 = jnp.minimum(t + 1, _T - 1)
        t2 = jnp.minimum(t + 2, _T - 1)
        corr_n = _dot(tsn_s[pl.ds(t1, 1), :],
                      jnp.broadcast_to(new_row, (8, _D)))
        nn = jnp.sqrt(jnp.sum(new_row * new_row))
        hp = hstar + 1.0
        s1 = s + 1.0
        onehot = flat == n_star
        nrm = jnp.where(onehot, nn, nrm_s[...])
        nrm_s[...] = nrm
        h = jnp.where(onehot, hp, h_s[...])
        h_s[...] = h
        dot_n2 = _dot(tsn_s[pl.ds(t2, 1), :], W_s[...]).reshape(8, 128)
        beta1n = dot_stale / nrm
        gain1n = jnp.exp(_LAM * (1.0 - 1024.0 * h / s1))
        score1n = gain1n * beta1n
        masked = jnp.where(onehot, neg, score1n)
        m1n = jnp.max(masked)
        i1n = jnp.argmax(masked).astype(jnp.int32)
        oh1n = flat == i1n
        bstar1n = jnp.sum(jnp.where(oh1n, beta1n, 0.0))
        hstar1n = jnp.sum(jnp.where(oh1n, h, 0.0))
        gpn = jnp.sum(jnp.where(onehot, gain1n, 0.0))
        Ck1n = W_s[pl.ds(i1n, 1), :]
        return (s1, n_star, nn, hp, gpn, new_row, corr_n,
                m1n, i1n, bstar1n, hstar1n, Ck1n, dot_n2)

    carry = (s0, jnp.int32(-1), 1.0, 1.0, 1.0,
             jnp.zeros((1, _D), jnp.float32), corr0,
             m1, i1, bstar1, hstar1, Ck1, dot1)
    out = jax.lax.fori_loop(0, _T, step, carry, unroll=2)
    out_ref[0, 0] = out[1]


def kernel(all_ts, W, cumhisto):
    h2 = cumhisto.reshape(8, 128)
    out = pl.pallas_call(
        _body,
        out_shape=jax.ShapeDtypeStruct((1, 1), jnp.int32),
        in_specs=[
            pl.BlockSpec(memory_space=pltpu.VMEM),
            pl.BlockSpec(memory_space=pltpu.VMEM),
            pl.BlockSpec(memory_space=pltpu.VMEM),
        ],
        out_specs=pl.BlockSpec(memory_space=pltpu.SMEM),
        scratch_shapes=[
            pltpu.VMEM((_T, _D), jnp.float32),
            pltpu.VMEM((_N, _D), jnp.float32),
            pltpu.VMEM((8, 128), jnp.float32),
            pltpu.VMEM((8, 128), jnp.float32),
        ],
    )(all_ts, W, h2)
    return out[0, 0]


# speedup vs baseline: 1.0839x; 1.0015x over previous
import jax
import jax.numpy as jnp
from jax.experimental import pallas as pl
from jax.experimental.pallas import tpu as pltpu

_N = 1024
_D = 256
_T = 4096
_LAM = 0.25


def _dot(lhs, rhs):
    return jax.lax.dot_general(
        lhs, rhs, (((1,), (1,)), ((), ())),
        preferred_element_type=jnp.float32)


def _body(ts_ref, W_ref, h_ref, out_ref, tsn_s, W_s, h_s, nrm_s):
    W_s[...] = W_ref[...]
    h_s[...] = h_ref[...]
    ts = ts_ref[...]
    tsn_s[...] = ts / jnp.sqrt(jnp.sum(ts * ts, axis=1, keepdims=True))
    Wv = W_s[...]
    nrm_s[...] = jnp.sqrt(jnp.sum(Wv * Wv, axis=1)).reshape(8, 128)

    ridx = jax.lax.broadcasted_iota(jnp.int32, (8, 128), 0)
    cidx = jax.lax.broadcasted_iota(jnp.int32, (8, 128), 1)
    flat = ridx * 128 + cidx
    neg = jnp.float32(-jnp.inf)

    s0 = jnp.sum(h_ref[...])
    h0 = h_s[...]
    nrm0 = nrm_s[...]
    beta1 = _dot(tsn_s[pl.ds(0, 1), :], W_s[...]).reshape(8, 128) / nrm0
    gain1 = jnp.exp(_LAM * (1.0 - 1024.0 * h0 / s0))
    score1 = gain1 * beta1
    m1 = jnp.max(score1)
    i1 = jnp.argmax(score1).astype(jnp.int32)
    oh1 = flat == i1
    bstar1 = jnp.sum(jnp.where(oh1, beta1, 0.0))
    hstar1 = jnp.sum(jnp.where(oh1, h0, 0.0))
    Ck1 = W_s[pl.ds(i1, 1), :]
    dot1 = _dot(tsn_s[pl.ds(1, 1), :], W_s[...]).reshape(8, 128)
    corr0 = jnp.full((1, 8), neg, jnp.float32)

    def step(t, carry):
        (s, r_prev, nn_prev, hp_prev, gp, npr, corr,
         m1, i1, bstar1, hstar1, Ck1, dot_stale) = carry
        c = jnp.max(corr)
        beta_p = c / nn_prev
        score_p = gp * beta_p
        take_prev = (score_p > m1) | ((score_p == m1) & (r_prev < i1))
        n_star = jnp.where(take_prev, r_prev, i1)
        bstar = jnp.where(take_prev, beta_p, bstar1)
        hstar = jnp.where(take_prev, hp_prev, hstar1)
        alpha = 0.01 / (1.0 + hstar / 20000.0)
        tsn = tsn_s[pl.ds(t, 1), :]
        Ck = jnp.where(take_prev, npr, Ck1)
        new_row = Ck + alpha * bstar * (tsn - Ck)
        W_s[pl.ds(n_star, 1), :] = new_row
        t1 = jnp.minimum(t + 1, _T - 1)
        t2 = jnp.minimum(t + 2, _T - 1)
        corr_n = _dot(tsn_s[pl.ds(t1, 1), :],
                      jnp.broadcast_to(new_row, (8, _D)))
        dot_n2 = _dot(tsn_s[pl.ds(t2, 1), :], W_s[...]).reshape(8, 128)
        nn = jnp.sqrt(jnp.sum(new_row * new_row))
        hp = hstar + 1.0
        s1 = s + 1.0
        onehot = flat == n_star
        nrm = jnp.where(onehot, nn, nrm_s[...])
        nrm_s[...] = nrm
        h = jnp.where(onehot, hp, h_s[...])
        h_s[...] = h
        beta1n = dot_stale / nrm
        gain1n = jnp.exp(_LAM * (1.0 - 1024.0 * h / s1))
        score1n = gain1n * beta1n
        masked = jnp.where(onehot, neg, score1n)
        m1n = jnp.max(masked)
        i1n = jnp.argmax(masked).astype(jnp.int32)
        oh1n = flat == i1n
        bstar1n = jnp.sum(jnp.where(oh1n, beta1n, 0.0))
        hstar1n = jnp.sum(jnp.where(oh1n, h, 0.0))
        gpn = jnp.sum(jnp.where(onehot, gain1n, 0.0))
        Ck1n = W_s[pl.ds(i1n, 1), :]
        return (s1, n_star, nn, hp, gpn, new_row, corr_n,
                m1n, i1n, bstar1n, hstar1n, Ck1n, dot_n2)

    carry = (s0, jnp.int32(-1), 1.0, 1.0, 1.0,
             jnp.zeros((1, _D), jnp.float32), corr0,
             m1, i1, bstar1, hstar1, Ck1, dot1)
    out = jax.lax.fori_loop(0, _T, step, carry, unroll=2)
    out_ref[0, 0] = out[1]


def kernel(all_ts, W, cumhisto):
    h2 = cumhisto.reshape(8, 128)
    out = pl.pallas_call(
        _body,
        out_shape=jax.ShapeDtypeStruct((1, 1), jnp.int32),
        in_specs=[
            pl.BlockSpec(memory_space=pltpu.VMEM),
            pl.BlockSpec(memory_space=pltpu.VMEM),
            pl.BlockSpec(memory_space=pltpu.VMEM),
        ],
        out_specs=pl.BlockSpec(memory_space=pltpu.SMEM),
        scratch_shapes=[
            pltpu.VMEM((_T, _D), jnp.float32),
            pltpu.VMEM((_N, _D), jnp.float32),
            pltpu.VMEM((8, 128), jnp.float32),
            pltpu.VMEM((8, 128), jnp.float32),
        ],
    )(all_ts, W, h2)
    return out[0, 0]


# speedup vs baseline: 1.1381x; 1.0501x over previous
import jax
import jax.numpy as jnp
from jax.experimental import pallas as pl
from jax.experimental.pallas import tpu as pltpu

_N = 1024
_D = 256
_T = 4096
_LAM = 0.25


def _dot(lhs, rhs):
    return jax.lax.dot_general(
        lhs, rhs, (((1,), (1,)), ((), ())),
        preferred_element_type=jnp.float32)


def _body(ts_ref, W_ref, h_ref, out_ref, tsn_s, W_s, h_s, nrm_s):
    W_s[...] = W_ref[...]
    h_s[...] = h_ref[...]
    ts = ts_ref[...]
    tsn_s[...] = ts / jnp.sqrt(jnp.sum(ts * ts, axis=1, keepdims=True))
    Wv = W_s[...]
    nrm_s[...] = jnp.sqrt(jnp.sum(Wv * Wv, axis=1)).reshape(8, 128)

    ridx = jax.lax.broadcasted_iota(jnp.int32, (8, 128), 0)
    cidx = jax.lax.broadcasted_iota(jnp.int32, (8, 128), 1)
    flat = ridx * 128 + cidx
    neg = jnp.float32(-jnp.inf)

    s0 = jnp.sum(h_ref[...])
    h0 = h_s[...]
    nrm0 = nrm_s[...]
    beta1 = _dot(tsn_s[pl.ds(0, 1), :], W_s[...]).reshape(8, 128) / nrm0
    gain1 = jnp.exp(_LAM * (1.0 - 1024.0 * h0 / s0))
    score1 = gain1 * beta1
    m1 = jnp.max(score1)
    i1 = jnp.argmax(score1).astype(jnp.int32)
    oh1 = flat == i1
    bstar1 = jnp.sum(jnp.where(oh1, beta1, 0.0))
    hstar1 = jnp.sum(jnp.where(oh1, h0, 0.0))
    Ck1 = W_s[pl.ds(i1, 1), :]
    dot1 = _dot(tsn_s[pl.ds(1, 1), :], W_s[...]).reshape(8, 128)
    corr0 = jnp.full((1, 8), neg, jnp.float32)

    def step(t, carry):
        (s, r_prev, nn_prev, hp_prev, gp, npr, corr,
         m1, i1, bstar1, hstar1, Ck1, dot_stale) = carry
        c = jnp.max(corr)
        beta_p = c / nn_prev
        score_p = gp * beta_p
        take_prev = (score_p > m1) | ((score_p == m1) & (r_prev < i1))
        n_star = jnp.where(take_prev, r_prev, i1)
        bstar = jnp.where(take_prev, beta_p, bstar1)
        hstar = jnp.where(take_prev, hp_prev, hstar1)
        alpha = 0.01 / (1.0 + hstar / 20000.0)
        tsn = tsn_s[pl.ds(t, 1), :]
        Ck = jnp.where(take_prev, npr, Ck1)
        new_row = Ck + alpha * bstar * (tsn - Ck)
        W_s[pl.ds(n_star, 1), :] = new_row
        t1 = jnp.minimum(t + 1, _T - 1)
        t2 = jnp.minimum(t + 2, _T - 1)
        corr_n = _dot(tsn_s[pl.ds(t1, 1), :],
                      jnp.broadcast_to(new_row, (8, _D)))
        dot_n2 = _dot(tsn_s[pl.ds(t2, 1), :], W_s[...]).reshape(8, 128)
        nn = jnp.sqrt(jnp.sum(new_row * new_row))
        hp = hstar + 1.0
        s1 = s + 1.0
        onehot = flat == n_star
        nrm = jnp.where(onehot, nn, nrm_s[...])
        nrm_s[...] = nrm
        h = jnp.where(onehot, hp, h_s[...])
        h_s[...] = h
        beta1n = dot_stale / nrm
        gain1n = jnp.exp(_LAM * (1.0 - 1024.0 * h / s1))
        score1n = gain1n * beta1n
        masked = jnp.where(onehot, neg, score1n)
        m1n = jnp.max(masked)
        i1n = jnp.argmax(masked).astype(jnp.int32)
        oh1n = flat == i1n
        bstar1n = jnp.sum(jnp.where(oh1n, beta1n, 0.0))
        hstar1n = jnp.sum(jnp.where(oh1n, h, 0.0))
        gpn = jnp.sum(jnp.where(onehot, gain1n, 0.0))
        Ck1n = W_s[pl.ds(i1n, 1), :]
        return (s1, n_star, nn, hp, gpn, new_row, corr_n,
                m1n, i1n, bstar1n, hstar1n, Ck1n, dot_n2)

    carry = (s0, jnp.int32(-1), 1.0, 1.0, 1.0,
             jnp.zeros((1, _D), jnp.float32), corr0,
             m1, i1, bstar1, hstar1, Ck1, dot1)
    out = jax.lax.fori_loop(0, _T, step, carry, unroll=4)
    out_ref[0, 0] = out[1]


def kernel(all_ts, W, cumhisto):
    h2 = cumhisto.reshape(8, 128)
    out = pl.pallas_call(
        _body,
        out_shape=jax.ShapeDtypeStruct((1, 1), jnp.int32),
        in_specs=[
            pl.BlockSpec(memory_space=pltpu.VMEM),
            pl.BlockSpec(memory_space=pltpu.VMEM),
            pl.BlockSpec(memory_space=pltpu.VMEM),
        ],
        out_specs=pl.BlockSpec(memory_space=pltpu.SMEM),
        scratch_shapes=[
            pltpu.VMEM((_T, _D), jnp.float32),
            pltpu.VMEM((_N, _D), jnp.float32),
            pltpu.VMEM((8, 128), jnp.float32),
            pltpu.VMEM((8, 128), jnp.float32),
        ],
    )(all_ts, W, h2)
    return out[0, 0]


# speedup vs baseline: 1.1644x; 1.0230x over previous
import jax
import jax.numpy as jnp
from jax.experimental import pallas as pl
from jax.experimental.pallas import tpu as pltpu

_N = 1024
_D = 256
_T = 4096
_LAM = 0.25


def _dot(lhs, rhs):
    return jax.lax.dot_general(
        lhs, rhs, (((1,), (1,)), ((), ())),
        preferred_element_type=jnp.float32)


def _body(ts_ref, W_ref, h_ref, out_ref, tsn_s, W_s, h_s, nrm_s):
    W_s[...] = W_ref[...]
    h_s[...] = h_ref[...]
    ts = ts_ref[...]
    tsn_s[...] = ts / jnp.sqrt(jnp.sum(ts * ts, axis=1, keepdims=True))
    Wv = W_s[...]
    nrm_s[...] = jnp.sqrt(jnp.sum(Wv * Wv, axis=1)).reshape(8, 128)

    ridx = jax.lax.broadcasted_iota(jnp.int32, (8, 128), 0)
    cidx = jax.lax.broadcasted_iota(jnp.int32, (8, 128), 1)
    flat = ridx * 128 + cidx
    neg = jnp.float32(-jnp.inf)

    s0 = jnp.sum(h_ref[...])
    h0 = h_s[...]
    nrm0 = nrm_s[...]
    beta1 = _dot(tsn_s[pl.ds(0, 1), :], W_s[...]).reshape(8, 128) / nrm0
    gain1 = jnp.exp(_LAM * (1.0 - 1024.0 * h0 / s0))
    score1 = gain1 * beta1
    m1 = jnp.max(score1)
    i1 = jnp.argmax(score1).astype(jnp.int32)
    oh1 = flat == i1
    bstar1 = jnp.sum(jnp.where(oh1, beta1, 0.0))
    hstar1 = jnp.sum(jnp.where(oh1, h0, 0.0))
    Ck1 = W_s[pl.ds(i1, 1), :]
    dot1 = _dot(tsn_s[pl.ds(1, 1), :], W_s[...]).reshape(8, 128)
    corr0 = jnp.full((1, 8), neg, jnp.float32)

    def step(t, carry):
        (s, r_prev, nn_prev, hp_prev, gp, npr, corr,
         m1, i1, bstar1, hstar1, Ck1, dot_stale) = carry
        c = jnp.max(corr)
        beta_p = c / nn_prev
        score_p = gp * beta_p
        take_prev = (score_p > m1) | ((score_p == m1) & (r_prev < i1))
        n_star = jnp.where(take_prev, r_prev, i1)
        bstar = jnp.where(take_prev, beta_p, bstar1)
        hstar = jnp.where(take_prev, hp_prev, hstar1)
        alpha = 0.01 / (1.0 + hstar / 20000.0)
        tsn = tsn_s[pl.ds(t, 1), :]
        Ck = jnp.where(take_prev, npr, Ck1)
        new_row = Ck + alpha * bstar * (tsn - Ck)
        W_s[pl.ds(n_star, 1), :] = new_row
        t1 = jnp.minimum(t + 1, _T - 1)
        t2 = jnp.minimum(t + 2, _T - 1)
        corr_n = _dot(tsn_s[pl.ds(t1, 1), :],
                      jnp.broadcast_to(new_row, (8, _D)))
        dot_n2 = _dot(tsn_s[pl.ds(t2, 1), :], W_s[...]).reshape(8, 128)
        nn = jnp.sqrt(jnp.sum(new_row * new_row))
        hp = hstar + 1.0
        s1 = s + 1.0
        onehot = flat == n_star
        nrm = jnp.where(onehot, nn, nrm_s[...])
        nrm_s[...] = nrm
        h = jnp.where(onehot, hp, h_s[...])
        h_s[...] = h
        beta1n = dot_stale / nrm
        gain1n = jnp.exp(_LAM * (1.0 - 1024.0 * h / s1))
        score1n = gain1n * beta1n
        masked = jnp.where(onehot, neg, score1n)
        m1n = jnp.max(masked)
        i1n = jnp.argmax(masked).astype(jnp.int32)
        oh1n = flat == i1n
        bstar1n = jnp.sum(jnp.where(oh1n, beta1n, 0.0))
        hstar1n = jnp.sum(jnp.where(oh1n, h, 0.0))
        gpn = jnp.sum(jnp.where(onehot, gain1n, 0.0))
        Ck1n = W_s[pl.ds(i1n, 1), :]
        return (s1, n_star, nn, hp, gpn, new_row, corr_n,
                m1n, i1n, bstar1n, hstar1n, Ck1n, dot_n2)

    carry = (s0, jnp.int32(-1), 1.0, 1.0, 1.0,
             jnp.zeros((1, _D), jnp.float32), corr0,
             m1, i1, bstar1, hstar1, Ck1, dot1)
    out = jax.lax.fori_loop(0, _T, step, carry, unroll=8)
    out_ref[0, 0] = out[1]


def kernel(all_ts, W, cumhisto):
    h2 = cumhisto.reshape(8, 128)
    out = pl.pallas_call(
        _body,
        out_shape=jax.ShapeDtypeStruct((1, 1), jnp.int32),
        in_specs=[
            pl.BlockSpec(memory_space=pltpu.VMEM),
            pl.BlockSpec(memory_space=pltpu.VMEM),
            pl.BlockSpec(memory_space=pltpu.VMEM),
        ],
        out_specs=pl.BlockSpec(memory_space=pltpu.SMEM),
        scratch_shapes=[
            pltpu.VMEM((_T, _D), jnp.float32),
            pltpu.VMEM((_N, _D), jnp.float32),
            pltpu.VMEM((8, 128), jnp.float32),
            pltpu.VMEM((8, 128), jnp.float32),
        ],
    )(all_ts, W, h2)
    return out[0, 0]
